# baseline (speedup 1.0000x reference)
import functools

import jax
import jax.numpy as jnp
from jax import lax
from jax.experimental import pallas as pl
from jax.experimental.pallas import tpu as pltpu
from jax.experimental.pallas import tpu_sc as plsc

NC = 2
NS = 16
NW = NC * NS

CH = 128
CS = 64
JUNK = 1024


def _sc_mesh():
    return plsc.VectorSubcoreMesh(core_axis_name="c", subcore_axis_name="s")


def _pad_edges(src, dst, N, E):
    ep = NW * CH * 8
    E_pad = ((E + ep - 1) // ep) * ep
    pad = E_pad - E
    src_p = jnp.concatenate([src, jnp.full((pad,), N, jnp.int32)])
    ar = jnp.arange(pad, dtype=jnp.int32)
    dst_deg = jnp.concatenate([dst, N + ar % JUNK])
    dst_scat = jnp.concatenate([dst, ar % 8192])
    return (src_p.reshape(E_pad // CH, CH), dst_deg.reshape(E_pad // CH, CH),
            dst_scat.reshape(E_pad // CH, CH), E_pad)


def _zero_acc(zeros_hbm, acc, sid, na):
    nzt, rz = na // 1000, 1000
    tail = na - (na // 1000) * 1000

    @pl.when(sid < nzt)
    def _():
        pltpu.sync_copy(zeros_hbm.at[pl.ds(sid * rz, rz)],
                        acc.at[pl.ds(sid * rz, rz)])

    if tail:
        @pl.when(sid == nzt)
        def _():
            pltpu.sync_copy(zeros_hbm.at[pl.ds(nzt * rz, tail)],
                            acc.at[pl.ds(nzt * rz, tail)])


def _dump_acc(acc, out_hbm, c, sid, n):
    ndt, rd = n // 1000, 1000

    @pl.when(sid < ndt)
    def _():
        pltpu.sync_copy(acc.at[pl.ds(sid * rd, rd)],
                        out_hbm.at[c, pl.ds(sid * rd, rd)])


def _make_deg_kernel(E_pad, N):
    per_tile = E_pad // NW
    nch = per_tile // CH
    na = N + JUNK

    @functools.partial(
        pl.kernel,
        out_type=jax.ShapeDtypeStruct((NC, N, CH), jnp.float32),
        mesh=_sc_mesh(),
        scratch_types=[
            pltpu.VMEM((nch, CH), jnp.int32),
            pltpu.VMEM((CH, CH), jnp.float32),
            pltpu.VMEM_SHARED((na, CH), jnp.float32),
            pltpu.SemaphoreType.DMA,
        ],
    )
    def deg_kernel(dst2_hbm, ones_hbm, zeros_hbm, out_hbm,
                   dst_v, ones_v, acc, sem):
        c = lax.axis_index("c")
        sid = lax.axis_index("s")
        wid = sid * NC + c
        pltpu.sync_copy(dst2_hbm.at[pl.ds(wid * nch, nch)], dst_v)
        pltpu.sync_copy(ones_hbm, ones_v)
        _zero_acc(zeros_hbm, acc, sid, na)
        plsc.subcore_barrier()

        def fire(j, carry):
            pltpu.async_copy(ones_v, acc.at[dst_v.at[j]], sem, add=True)
            return carry

        lax.fori_loop(0, nch, fire, 0)

        def drain(j, carry):
            pltpu.make_async_copy(ones_v, acc.at[dst_v.at[0]], sem).wait()
            return carry

        lax.fori_loop(0, nch, drain, 0)
        plsc.subcore_barrier()
        _dump_acc(acc, out_hbm, c, sid, N)

    return deg_kernel


def _make_scatter_kernel(E_pad, N, H):
    per_tile = E_pad // NW
    nch = per_tile // CH
    nh = nch // 2

    @functools.partial(
        pl.kernel,
        out_type=jax.ShapeDtypeStruct((NC, N, H), jnp.float32),
        mesh=_sc_mesh(),
        scratch_types=[
            pltpu.VMEM((nh, CH), jnp.int32),
            pltpu.VMEM((nh, CH), jnp.int32),
            pltpu.VMEM((CH, H), jnp.float32),
            pltpu.VMEM((CH, H), jnp.float32),
            pltpu.VMEM_SHARED((N, H), jnp.float32),
            pltpu.SemaphoreType.DMA,
            pltpu.SemaphoreType.DMA,
        ],
    )
    def scat_kernel(p_hbm, src2_hbm, dst2_hbm, zeros_hbm, out_hbm,
                    src_v, dst_v, r0, r1, acc, g0, g1):
        c = lax.axis_index("c")
        sid = lax.axis_index("s")
        wid = sid * NC + c
        rows = (r0, r1)
        gsem = (g0, g1)

        _zero_acc(zeros_hbm, acc, sid, N)
        plsc.subcore_barrier()

        def gfire(k, b):
            pltpu.async_copy(p_hbm.at[src_v.at[k]], rows[b], gsem[b])

        def gwait(b):
            pltpu.make_async_copy(p_hbm.at[src_v.at[0]],
                                  rows[b], gsem[b]).wait()

        for h in range(2):
            pltpu.sync_copy(src2_hbm.at[pl.ds(wid * nch + h * nh, nh)], src_v)
            pltpu.sync_copy(dst2_hbm.at[pl.ds(wid * nch + h * nh, nh)], dst_v)
            gfire(0, 0)

            def body(t, carry):
                for b in range(2):
                    k = 2 * t + b
                    b2 = 1 - b
                    if b == 0:
                        gfire(k + 1, b2)
                    else:
                        @pl.when(t < nh // 2 - 1)
                        def _():
                            gfire(k + 1, b2)

                    gwait(b)
                    pltpu.sync_copy(rows[b], acc.at[dst_v.at[k]], add=True)
                return carry

            lax.fori_loop(0, nh // 2, body, 0)

        plsc.subcore_barrier()
        _dump_acc(acc, out_hbm, c, sid, N)

    return scat_kernel


def _tc_k1(x, W1, dega, degb, B):
    N, F = x.shape
    H = W1.shape[1]
    grid = N // B

    def body(x_ref, w_ref, da_ref, db_ref, p_ref, dinv_ref):
        deg = da_ref[...] + db_ref[...] + 1.0
        dinv = lax.rsqrt(deg)
        z = jnp.dot(x_ref[...], w_ref[...], preferred_element_type=jnp.float32)
        p_ref[...] = dinv * z
        dinv_ref[...] = dinv

    return pl.pallas_call(
        body,
        grid=(grid,),
        in_specs=[
            pl.BlockSpec((B, F), lambda i: (i, 0)),
            pl.BlockSpec((F, H), lambda i: (0, 0)),
            pl.BlockSpec((B, 1), lambda i: (i, 0)),
            pl.BlockSpec((B, 1), lambda i: (i, 0)),
        ],
        out_specs=[
            pl.BlockSpec((B, H), lambda i: (i, 0)),
            pl.BlockSpec((B, 1), lambda i: (i, 0)),
        ],
        out_shape=[
            jax.ShapeDtypeStruct((N, H), jnp.float32),
            jax.ShapeDtypeStruct((N, 1), jnp.float32),
        ],
    )(x, W1, dega, degb)


def _tc_k2(Sa, Sb, p1, dinv, b1, W2, B):
    N, H = p1.shape
    grid = N // B

    def body(sa_ref, sb_ref, p_ref, dinv_ref, b1_ref, w2_ref, p2_ref):
        agg = sa_ref[...] + sb_ref[...] + p_ref[...]
        h1 = jnp.maximum(dinv_ref[...] * agg + b1_ref[...], 0.0)
        z2 = jnp.dot(h1, w2_ref[...], preferred_element_type=jnp.float32)
        p2_ref[...] = dinv_ref[...] * z2

    return pl.pallas_call(
        body,
        grid=(grid,),
        in_specs=[
            pl.BlockSpec((B, H), lambda i: (i, 0)),
            pl.BlockSpec((B, H), lambda i: (i, 0)),
            pl.BlockSpec((B, H), lambda i: (i, 0)),
            pl.BlockSpec((B, 1), lambda i: (i, 0)),
            pl.BlockSpec((1, H), lambda i: (0, 0)),
            pl.BlockSpec((H, H), lambda i: (0, 0)),
        ],
        out_specs=pl.BlockSpec((B, H), lambda i: (i, 0)),
        out_shape=jax.ShapeDtypeStruct((N, H), jnp.float32),
    )(Sa, Sb, p1, dinv, b1, W2)


def _tc_k3(Sa, Sb, p2, dinv, b2, batch2, Wl, bl, G, B):
    N, H = p2.shape
    C = Wl.shape[1]
    grid = N // B

    def body(sa_ref, sb_ref, p_ref, dinv_ref, b2_ref, bat_ref, wl_ref, bl_ref,
             out_ref, pooled):
        i = pl.program_id(0)
        agg = sa_ref[...] + sb_ref[...] + p_ref[...]
        h2 = jnp.maximum(dinv_ref[...] * agg + b2_ref[...], 0.0)
        gids = lax.broadcasted_iota(jnp.int32, (B, G), 1)
        onehot = (bat_ref[...] == gids).astype(jnp.float32)
        blk = lax.dot_general(onehot, h2, (((0,), (0,)), ((), ())),
                              preferred_element_type=jnp.float32)

        @pl.when(i == 0)
        def _():
            pooled[...] = blk

        @pl.when(i > 0)
        def _():
            pooled[...] = pooled[...] + blk

        @pl.when(i == grid - 1)
        def _():
            out_ref[...] = jnp.dot(pooled[...], wl_ref[...],
                                   preferred_element_type=jnp.float32) + bl_ref[...]

    return pl.pallas_call(
        body,
        grid=(grid,),
        in_specs=[
            pl.BlockSpec((B, H), lambda i: (i, 0)),
            pl.BlockSpec((B, H), lambda i: (i, 0)),
            pl.BlockSpec((B, H), lambda i: (i, 0)),
            pl.BlockSpec((B, 1), lambda i: (i, 0)),
            pl.BlockSpec((1, H), lambda i: (0, 0)),
            pl.BlockSpec((B, 1), lambda i: (i, 0)),
            pl.BlockSpec((H, C), lambda i: (0, 0)),
            pl.BlockSpec((1, C), lambda i: (0, 0)),
        ],
        out_specs=pl.BlockSpec((G, C), lambda i: (0, 0)),
        out_shape=jax.ShapeDtypeStruct((G, C), jnp.float32),
        scratch_shapes=[pltpu.VMEM((G, H), jnp.float32)],
    )(Sa, Sb, p2, dinv, b2, batch2, Wl, bl)


@jax.jit
def kernel(x, edge_index, batch, W1, b1, W2, b2, Wl, bl):
    N, F = x.shape
    H = W1.shape[1]
    E = edge_index.shape[1]
    G = 64
    B = 1000

    src_p, dst2_deg, dst2_scat, E_pad = _pad_edges(
        edge_index[0], edge_index[1], N, E)
    ones_c = jnp.ones((CH, CH), jnp.float32)
    zerosA = jnp.zeros((N + JUNK, CH), jnp.float32)
    zrow = jnp.zeros((8, H), jnp.float32)

    deg2 = _make_deg_kernel(E_pad, N)(dst2_deg, ones_c, zerosA)
    dega, degb = deg2[0, :, 0:1], deg2[1, :, 0:1]

    p1, dinv = _tc_k1(x, W1, dega, degb, B)

    scat = _make_scatter_kernel(E_pad, N, H)
    S1 = scat(jnp.concatenate([p1, zrow]), src_p, dst2_scat, zerosA)
    p2 = _tc_k2(S1[0], S1[1], p1, dinv, b1.reshape(1, H), W2, B)

    S2 = scat(jnp.concatenate([p2, zrow]), src_p, dst2_scat, zerosA)
    return _tc_k3(S2[0], S2[1], p2, dinv, b2.reshape(1, H),
                  batch.reshape(N, 1), Wl, bl.reshape(1, Wl.shape[1]), G, B)


# speedup vs baseline: 1.4009x; 1.4009x over previous
import functools

import jax
import jax.numpy as jnp
from jax import lax
from jax.experimental import pallas as pl
from jax.experimental.pallas import tpu as pltpu
from jax.experimental.pallas import tpu_sc as plsc

NC = 2
NS = 16
NW = NC * NS

CH = 128
CS = 64
JUNK = 1024


def _sc_mesh():
    return plsc.VectorSubcoreMesh(core_axis_name="c", subcore_axis_name="s")


def _pad_edges(src, dst, N, E):
    ep = NW * CH * 8
    E_pad = ((E + ep - 1) // ep) * ep
    pad = E_pad - E
    src_p = jnp.concatenate([src, jnp.full((pad,), N, jnp.int32)])
    ar = jnp.arange(pad, dtype=jnp.int32)
    dst_deg = jnp.concatenate([dst, N + ar % JUNK])
    dst_scat = jnp.concatenate([dst, ar % 8192])
    return (src_p.reshape(E_pad // CH, CH), dst_deg.reshape(E_pad // CH, CH),
            dst_scat.reshape(E_pad // CH, CH), E_pad)


def _zero_acc(zeros_hbm, acc, sid, na):
    nzt, rz = na // 1000, 1000
    tail = na - (na // 1000) * 1000

    @pl.when(sid < nzt)
    def _():
        pltpu.sync_copy(zeros_hbm.at[pl.ds(sid * rz, rz)],
                        acc.at[pl.ds(sid * rz, rz)])

    if tail:
        @pl.when(sid == nzt)
        def _():
            pltpu.sync_copy(zeros_hbm.at[pl.ds(nzt * rz, tail)],
                            acc.at[pl.ds(nzt * rz, tail)])


def _dump_acc(acc, out_hbm, c, sid, n):
    ndt, rd = n // 1000, 1000

    @pl.when(sid < ndt)
    def _():
        pltpu.sync_copy(acc.at[pl.ds(sid * rd, rd)],
                        out_hbm.at[c, pl.ds(sid * rd, rd)])


def _make_deg_kernel(E_pad, N):
    per_tile = E_pad // NW
    nch = per_tile // CH
    na = N + JUNK

    @functools.partial(
        pl.kernel,
        out_type=jax.ShapeDtypeStruct((NC, N, CH), jnp.float32),
        mesh=_sc_mesh(),
        scratch_types=[
            pltpu.VMEM((nch, CH), jnp.int32),
            pltpu.VMEM((CH, CH), jnp.float32),
            pltpu.VMEM_SHARED((na, CH), jnp.float32),
            pltpu.SemaphoreType.DMA,
        ],
    )
    def deg_kernel(dst2_hbm, ones_hbm, zeros_hbm, out_hbm,
                   dst_v, ones_v, acc, sem):
        c = lax.axis_index("c")
        sid = lax.axis_index("s")
        wid = sid * NC + c
        pltpu.sync_copy(dst2_hbm.at[pl.ds(wid * nch, nch)], dst_v)
        pltpu.sync_copy(ones_hbm, ones_v)
        _zero_acc(zeros_hbm, acc, sid, na)
        plsc.subcore_barrier()

        def fire(j, carry):
            pltpu.async_copy(ones_v, acc.at[dst_v.at[j]], sem, add=True)
            return carry

        lax.fori_loop(0, nch, fire, 0)

        def drain(j, carry):
            pltpu.make_async_copy(ones_v, acc.at[dst_v.at[0]], sem).wait()
            return carry

        lax.fori_loop(0, nch, drain, 0)
        plsc.subcore_barrier()
        _dump_acc(acc, out_hbm, c, sid, N)

    return deg_kernel


CHUNK = 80


def _make_scatter_kernel(E, N, H):
    per_tile = E // NW
    nch = per_tile // CHUNK

    @functools.partial(
        pl.kernel,
        out_type=jax.ShapeDtypeStruct((NC, N, H), jnp.float32),
        mesh=_sc_mesh(),
        scratch_types=[
            pltpu.VMEM((CHUNK,), jnp.int32),
            pltpu.VMEM((CHUNK,), jnp.int32),
            pltpu.VMEM((CHUNK, H), jnp.float32),
            pltpu.VMEM_SHARED((N, H), jnp.float32),
            pltpu.SemaphoreType.DMA,
        ],
    )
    def scat_kernel(p_hbm, src_hbm, dst_hbm, zeros_hbm, out_hbm,
                    src_v, dst_v, rows_v, acc, sem):
        c = lax.axis_index("c")
        sid = lax.axis_index("s")
        wid = sid * NC + c
        _zero_acc(zeros_hbm, acc, sid, N)
        plsc.subcore_barrier()
        base = wid * per_tile

        def body(j, carry):
            pltpu.sync_copy(src_hbm.at[pl.ds(base + j * CHUNK, CHUNK)], src_v)
            pltpu.sync_copy(dst_hbm.at[pl.ds(base + j * CHUNK, CHUNK)], dst_v)
            pltpu.async_copy(p_hbm.at[src_v], rows_v, sem).wait()
            pltpu.sync_copy(rows_v, acc.at[dst_v], add=True)
            return carry

        lax.fori_loop(0, nch, body, 0)
        plsc.subcore_barrier()
        _dump_acc(acc, out_hbm, c, sid, N)

    return scat_kernel


def _tc_k1(x, W1, dega, degb, B):
    N, F = x.shape
    H = W1.shape[1]
    grid = N // B

    def body(x_ref, w_ref, da_ref, db_ref, p_ref, dinv_ref):
        deg = da_ref[...] + db_ref[...] + 1.0
        dinv = lax.rsqrt(deg)
        z = jnp.dot(x_ref[...], w_ref[...], preferred_element_type=jnp.float32)
        p_ref[...] = dinv * z
        dinv_ref[...] = dinv

    return pl.pallas_call(
        body,
        grid=(grid,),
        in_specs=[
            pl.BlockSpec((B, F), lambda i: (i, 0)),
            pl.BlockSpec((F, H), lambda i: (0, 0)),
            pl.BlockSpec((B, 1), lambda i: (i, 0)),
            pl.BlockSpec((B, 1), lambda i: (i, 0)),
        ],
        out_specs=[
            pl.BlockSpec((B, H), lambda i: (i, 0)),
            pl.BlockSpec((B, 1), lambda i: (i, 0)),
        ],
        out_shape=[
            jax.ShapeDtypeStruct((N, H), jnp.float32),
            jax.ShapeDtypeStruct((N, 1), jnp.float32),
        ],
    )(x, W1, dega, degb)


def _tc_k2(Sa, Sb, p1, dinv, b1, W2, B):
    N, H = p1.shape
    grid = N // B

    def body(sa_ref, sb_ref, p_ref, dinv_ref, b1_ref, w2_ref, p2_ref):
        agg = sa_ref[...] + sb_ref[...] + p_ref[...]
        h1 = jnp.maximum(dinv_ref[...] * agg + b1_ref[...], 0.0)
        z2 = jnp.dot(h1, w2_ref[...], preferred_element_type=jnp.float32)
        p2_ref[...] = dinv_ref[...] * z2

    return pl.pallas_call(
        body,
        grid=(grid,),
        in_specs=[
            pl.BlockSpec((B, H), lambda i: (i, 0)),
            pl.BlockSpec((B, H), lambda i: (i, 0)),
            pl.BlockSpec((B, H), lambda i: (i, 0)),
            pl.BlockSpec((B, 1), lambda i: (i, 0)),
            pl.BlockSpec((1, H), lambda i: (0, 0)),
            pl.BlockSpec((H, H), lambda i: (0, 0)),
        ],
        out_specs=pl.BlockSpec((B, H), lambda i: (i, 0)),
        out_shape=jax.ShapeDtypeStruct((N, H), jnp.float32),
    )(Sa, Sb, p1, dinv, b1, W2)


def _tc_k3(Sa, Sb, p2, dinv, b2, batch2, Wl, bl, G, B):
    N, H = p2.shape
    C = Wl.shape[1]
    grid = N // B

    def body(sa_ref, sb_ref, p_ref, dinv_ref, b2_ref, bat_ref, wl_ref, bl_ref,
             out_ref, pooled):
        i = pl.program_id(0)
        agg = sa_ref[...] + sb_ref[...] + p_ref[...]
        h2 = jnp.maximum(dinv_ref[...] * agg + b2_ref[...], 0.0)
        gids = lax.broadcasted_iota(jnp.int32, (B, G), 1)
        onehot = (bat_ref[...] == gids).astype(jnp.float32)
        blk = lax.dot_general(onehot, h2, (((0,), (0,)), ((), ())),
                              preferred_element_type=jnp.float32)

        @pl.when(i == 0)
        def _():
            pooled[...] = blk

        @pl.when(i > 0)
        def _():
            pooled[...] = pooled[...] + blk

        @pl.when(i == grid - 1)
        def _():
            out_ref[...] = jnp.dot(pooled[...], wl_ref[...],
                                   preferred_element_type=jnp.float32) + bl_ref[...]

    return pl.pallas_call(
        body,
        grid=(grid,),
        in_specs=[
            pl.BlockSpec((B, H), lambda i: (i, 0)),
            pl.BlockSpec((B, H), lambda i: (i, 0)),
            pl.BlockSpec((B, H), lambda i: (i, 0)),
            pl.BlockSpec((B, 1), lambda i: (i, 0)),
            pl.BlockSpec((1, H), lambda i: (0, 0)),
            pl.BlockSpec((B, 1), lambda i: (i, 0)),
            pl.BlockSpec((H, C), lambda i: (0, 0)),
            pl.BlockSpec((1, C), lambda i: (0, 0)),
        ],
        out_specs=pl.BlockSpec((G, C), lambda i: (0, 0)),
        out_shape=jax.ShapeDtypeStruct((G, C), jnp.float32),
        scratch_shapes=[pltpu.VMEM((G, H), jnp.float32)],
    )(Sa, Sb, p2, dinv, b2, batch2, Wl, bl)


@jax.jit
def kernel(x, edge_index, batch, W1, b1, W2, b2, Wl, bl):
    N, F = x.shape
    H = W1.shape[1]
    E = edge_index.shape[1]
    G = 64
    B = 1000

    src, dst = edge_index[0], edge_index[1]
    _, dst2_deg, _, E_pad = _pad_edges(src, dst, N, E)
    ones_c = jnp.ones((CH, CH), jnp.float32)
    zerosA = jnp.zeros((N + JUNK, CH), jnp.float32)

    deg2 = _make_deg_kernel(E_pad, N)(dst2_deg, ones_c, zerosA)
    dega, degb = deg2[0, :, 0:1], deg2[1, :, 0:1]

    p1, dinv = _tc_k1(x, W1, dega, degb, B)

    scat = _make_scatter_kernel(E, N, H)
    S1 = scat(p1, src, dst, zerosA)
    p2 = _tc_k2(S1[0], S1[1], p1, dinv, b1.reshape(1, H), W2, B)

    S2 = scat(p2, src, dst, zerosA)
    return _tc_k3(S2[0], S2[1], p2, dinv, b2.reshape(1, H),
                  batch.reshape(N, 1), Wl, bl.reshape(1, Wl.shape[1]), G, B)


# speedup vs baseline: 1.8955x; 1.3530x over previous
import functools

import jax
import jax.numpy as jnp
from jax import lax
from jax.experimental import pallas as pl
from jax.experimental.pallas import tpu as pltpu
from jax.experimental.pallas import tpu_sc as plsc

NC = 2
NS = 16
NW = NC * NS

CH = 128
CS = 64
JUNK = 1024


def _sc_mesh():
    return plsc.VectorSubcoreMesh(core_axis_name="c", subcore_axis_name="s")


def _pad_edges(src, dst, N, E):
    ep = NW * CH * 8
    E_pad = ((E + ep - 1) // ep) * ep
    pad = E_pad - E
    src_p = jnp.concatenate([src, jnp.full((pad,), N, jnp.int32)])
    ar = jnp.arange(pad, dtype=jnp.int32)
    dst_deg = jnp.concatenate([dst, N + ar % JUNK])
    dst_scat = jnp.concatenate([dst, ar % 8192])
    return (src_p.reshape(E_pad // CH, CH), dst_deg.reshape(E_pad // CH, CH),
            dst_scat.reshape(E_pad // CH, CH), E_pad)


def _zero_acc(zeros_hbm, acc, sid, na):
    nzt, rz = na // 1000, 1000
    tail = na - (na // 1000) * 1000

    @pl.when(sid < nzt)
    def _():
        pltpu.sync_copy(zeros_hbm.at[pl.ds(sid * rz, rz)],
                        acc.at[pl.ds(sid * rz, rz)])

    if tail:
        @pl.when(sid == nzt)
        def _():
            pltpu.sync_copy(zeros_hbm.at[pl.ds(nzt * rz, tail)],
                            acc.at[pl.ds(nzt * rz, tail)])


def _dump_acc(acc, out_hbm, c, sid, n):
    ndt, rd = n // 1000, 1000

    @pl.when(sid < ndt)
    def _():
        pltpu.sync_copy(acc.at[pl.ds(sid * rd, rd)],
                        out_hbm.at[c, pl.ds(sid * rd, rd)])


def _make_deg_kernel(E_pad, N):
    per_tile = E_pad // NW
    nch = per_tile // CH
    na = N + JUNK

    @functools.partial(
        pl.kernel,
        out_type=jax.ShapeDtypeStruct((NC, N, CH), jnp.float32),
        mesh=_sc_mesh(),
        scratch_types=[
            pltpu.VMEM((nch, CH), jnp.int32),
            pltpu.VMEM((CH, CH), jnp.float32),
            pltpu.VMEM_SHARED((na, CH), jnp.float32),
            pltpu.SemaphoreType.DMA,
        ],
    )
    def deg_kernel(dst2_hbm, ones_hbm, zeros_hbm, out_hbm,
                   dst_v, ones_v, acc, sem):
        c = lax.axis_index("c")
        sid = lax.axis_index("s")
        wid = sid * NC + c
        pltpu.sync_copy(dst2_hbm.at[pl.ds(wid * nch, nch)], dst_v)
        pltpu.sync_copy(ones_hbm, ones_v)
        _zero_acc(zeros_hbm, acc, sid, na)
        plsc.subcore_barrier()

        def fire(j, carry):
            pltpu.async_copy(ones_v, acc.at[dst_v.at[j]], sem, add=True)
            return carry

        lax.fori_loop(0, nch, fire, 0)

        def drain(j, carry):
            pltpu.make_async_copy(ones_v, acc.at[dst_v.at[0]], sem).wait()
            return carry

        lax.fori_loop(0, nch, drain, 0)
        plsc.subcore_barrier()
        _dump_acc(acc, out_hbm, c, sid, N)

    return deg_kernel


CHUNK = 80


def _make_scatter_kernel(E, N, H):
    per_tile = E // NW
    nch = per_tile // CHUNK

    @functools.partial(
        pl.kernel,
        out_type=jax.ShapeDtypeStruct((NC, N, H), jnp.float32),
        mesh=_sc_mesh(),
        scratch_types=[
            pltpu.VMEM((CHUNK,), jnp.int32),
            pltpu.VMEM((CHUNK,), jnp.int32),
            pltpu.VMEM((CHUNK,), jnp.int32),
            pltpu.VMEM((CHUNK,), jnp.int32),
            pltpu.VMEM((CHUNK, H), jnp.float32),
            pltpu.VMEM_SHARED((N, H), jnp.float32),
            pltpu.SemaphoreType.DMA,
            pltpu.SemaphoreType.DMA,
            pltpu.SemaphoreType.DMA,
        ],
    )
    def scat_kernel(p_hbm, src_hbm, dst_hbm, zeros_hbm, out_hbm,
                    sv0, dv0, sv1, dv1, rows_v, acc, gsem, i0, i1):
        c = lax.axis_index("c")
        sid = lax.axis_index("s")
        wid = sid * NC + c
        srcs = (sv0, sv1)
        dsts = (dv0, dv1)
        isem = (i0, i1)
        _zero_acc(zeros_hbm, acc, sid, N)
        plsc.subcore_barrier()
        base = wid * per_tile

        def ifire(j, b):
            pltpu.async_copy(src_hbm.at[pl.ds(base + j * CHUNK, CHUNK)],
                             srcs[b], isem[b])
            pltpu.async_copy(dst_hbm.at[pl.ds(base + j * CHUNK, CHUNK)],
                             dsts[b], isem[b])

        def iwait(b):
            pltpu.make_async_copy(src_hbm.at[pl.ds(base, CHUNK)],
                                  srcs[b], isem[b]).wait()
            pltpu.make_async_copy(dst_hbm.at[pl.ds(base, CHUNK)],
                                  dsts[b], isem[b]).wait()

        ifire(0, 0)

        def body(t, carry):
            for b in range(2):
                j = 2 * t + b
                b2 = 1 - b
                ifire(j + 1, b2)
                iwait(b)
                pltpu.async_copy(p_hbm.at[srcs[b]], rows_v, gsem).wait()
                pltpu.sync_copy(rows_v, acc.at[dsts[b]], add=True)
            return carry

        lax.fori_loop(0, nch // 2, body, 0)
        iwait(0)
        pltpu.async_copy(p_hbm.at[srcs[0]], rows_v, gsem).wait()
        pltpu.sync_copy(rows_v, acc.at[dsts[0]], add=True)
        plsc.subcore_barrier()
        _dump_acc(acc, out_hbm, c, sid, N)

    return scat_kernel


def _tc_k1(x, W1, dega, degb, B):
    N, F = x.shape
    H = W1.shape[1]
    grid = N // B

    def body(x_ref, w_ref, da_ref, db_ref, p_ref, dinv_ref):
        deg = da_ref[...] + db_ref[...] + 1.0
        dinv = lax.rsqrt(deg)
        z = jnp.dot(x_ref[...], w_ref[...], preferred_element_type=jnp.float32)
        p_ref[...] = dinv * z
        dinv_ref[...] = dinv

    return pl.pallas_call(
        body,
        grid=(grid,),
        in_specs=[
            pl.BlockSpec((B, F), lambda i: (i, 0)),
            pl.BlockSpec((F, H), lambda i: (0, 0)),
            pl.BlockSpec((B, 1), lambda i: (i, 0)),
            pl.BlockSpec((B, 1), lambda i: (i, 0)),
        ],
        out_specs=[
            pl.BlockSpec((B, H), lambda i: (i, 0)),
            pl.BlockSpec((B, 1), lambda i: (i, 0)),
        ],
        out_shape=[
            jax.ShapeDtypeStruct((N, H), jnp.float32),
            jax.ShapeDtypeStruct((N, 1), jnp.float32),
        ],
    )(x, W1, dega, degb)


def _tc_k2(Sa, Sb, p1, dinv, b1, W2, B):
    N, H = p1.shape
    grid = N // B

    def body(sa_ref, sb_ref, p_ref, dinv_ref, b1_ref, w2_ref, p2_ref):
        agg = sa_ref[...] + sb_ref[...] + p_ref[...]
        h1 = jnp.maximum(dinv_ref[...] * agg + b1_ref[...], 0.0)
        z2 = jnp.dot(h1, w2_ref[...], preferred_element_type=jnp.float32)
        p2_ref[...] = dinv_ref[...] * z2

    return pl.pallas_call(
        body,
        grid=(grid,),
        in_specs=[
            pl.BlockSpec((B, H), lambda i: (i, 0)),
            pl.BlockSpec((B, H), lambda i: (i, 0)),
            pl.BlockSpec((B, H), lambda i: (i, 0)),
            pl.BlockSpec((B, 1), lambda i: (i, 0)),
            pl.BlockSpec((1, H), lambda i: (0, 0)),
            pl.BlockSpec((H, H), lambda i: (0, 0)),
        ],
        out_specs=pl.BlockSpec((B, H), lambda i: (i, 0)),
        out_shape=jax.ShapeDtypeStruct((N, H), jnp.float32),
    )(Sa, Sb, p1, dinv, b1, W2)


def _tc_k3(Sa, Sb, p2, dinv, b2, batch2, Wl, bl, G, B):
    N, H = p2.shape
    C = Wl.shape[1]
    grid = N // B

    def body(sa_ref, sb_ref, p_ref, dinv_ref, b2_ref, bat_ref, wl_ref, bl_ref,
             out_ref, pooled):
        i = pl.program_id(0)
        agg = sa_ref[...] + sb_ref[...] + p_ref[...]
        h2 = jnp.maximum(dinv_ref[...] * agg + b2_ref[...], 0.0)
        gids = lax.broadcasted_iota(jnp.int32, (B, G), 1)
        onehot = (bat_ref[...] == gids).astype(jnp.float32)
        blk = lax.dot_general(onehot, h2, (((0,), (0,)), ((), ())),
                              preferred_element_type=jnp.float32)

        @pl.when(i == 0)
        def _():
            pooled[...] = blk

        @pl.when(i > 0)
        def _():
            pooled[...] = pooled[...] + blk

        @pl.when(i == grid - 1)
        def _():
            out_ref[...] = jnp.dot(pooled[...], wl_ref[...],
                                   preferred_element_type=jnp.float32) + bl_ref[...]

    return pl.pallas_call(
        body,
        grid=(grid,),
        in_specs=[
            pl.BlockSpec((B, H), lambda i: (i, 0)),
            pl.BlockSpec((B, H), lambda i: (i, 0)),
            pl.BlockSpec((B, H), lambda i: (i, 0)),
            pl.BlockSpec((B, 1), lambda i: (i, 0)),
            pl.BlockSpec((1, H), lambda i: (0, 0)),
            pl.BlockSpec((B, 1), lambda i: (i, 0)),
            pl.BlockSpec((H, C), lambda i: (0, 0)),
            pl.BlockSpec((1, C), lambda i: (0, 0)),
        ],
        out_specs=pl.BlockSpec((G, C), lambda i: (0, 0)),
        out_shape=jax.ShapeDtypeStruct((G, C), jnp.float32),
        scratch_shapes=[pltpu.VMEM((G, H), jnp.float32)],
    )(Sa, Sb, p2, dinv, b2, batch2, Wl, bl)


@jax.jit
def kernel(x, edge_index, batch, W1, b1, W2, b2, Wl, bl):
    N, F = x.shape
    H = W1.shape[1]
    E = edge_index.shape[1]
    G = 64
    B = 1000

    src, dst = edge_index[0], edge_index[1]
    _, dst2_deg, _, E_pad = _pad_edges(src, dst, N, E)
    ones_c = jnp.ones((CH, CH), jnp.float32)
    zerosA = jnp.zeros((N + JUNK, CH), jnp.float32)

    deg2 = _make_deg_kernel(E_pad, N)(dst2_deg, ones_c, zerosA)
    dega, degb = deg2[0, :, 0:1], deg2[1, :, 0:1]

    p1, dinv = _tc_k1(x, W1, dega, degb, B)

    scat = _make_scatter_kernel(E, N, H)
    S1 = scat(p1, src, dst, zerosA)
    p2 = _tc_k2(S1[0], S1[1], p1, dinv, b1.reshape(1, H), W2, B)

    S2 = scat(p2, src, dst, zerosA)
    return _tc_k3(S2[0], S2[1], p2, dinv, b2.reshape(1, H),
                  batch.reshape(N, 1), Wl, bl.reshape(1, Wl.shape[1]), G, B)
